# Initial kernel scaffold; baseline (speedup 1.0000x reference)
import jax
import jax.numpy as jnp
from jax.experimental import pallas as pl


def kernel(categories, ce):
    raise NotImplementedError("write your pallas kernel here")


# speedup vs baseline: 3.2237x; 3.2237x over previous
import functools

import jax
import jax.numpy as jnp
from jax import lax
from jax.experimental import pallas as pl
from jax.experimental.pallas import tpu as pltpu
from jax.experimental.pallas import tpu_sc as plsc

MAX_LEN = 200
N_FILTERS = 128
BATCH = 4096
SEQ = 200

_B = BATCH * SEQ
_NC = 2
_NS = 16
_NW = _NC * _NS
_PER_W = _B // _NW
_R = 512
_CHUNKS = _PER_W // _R


def _build_gather():
    mesh = plsc.VectorSubcoreMesh(core_axis_name="c", subcore_axis_name="s")

    @functools.partial(
        pl.kernel,
        mesh=mesh,
        out_type=jax.ShapeDtypeStruct((_B, N_FILTERS), jnp.float32),
        scratch_types=[
            pltpu.VMEM((_R,), jnp.int32),
            pltpu.VMEM((_R, N_FILTERS), jnp.float32),
            pltpu.SemaphoreType.DMA,
        ],
    )
    def gather_kernel(table_hbm, idx_hbm, out_hbm, idx_v, rows_v, sem):
        wid = lax.axis_index("s") * _NC + lax.axis_index("c")
        base0 = wid * _PER_W

        def chunk(i, carry):
            base = base0 + i * _R
            pltpu.sync_copy(idx_hbm.at[pl.ds(base, _R)], idx_v)
            pltpu.async_copy(table_hbm.at[idx_v], rows_v, sem).wait()
            pltpu.sync_copy(rows_v, out_hbm.at[pl.ds(base, _R)])
            return carry

        lax.fori_loop(0, _CHUNKS, chunk, 0)

    return gather_kernel


_gather = _build_gather()


@jax.jit
def kernel(categories, ce):
    idx = categories.reshape(_B)
    out = _gather(ce, idx)
    return out.reshape(BATCH, SEQ, N_FILTERS)


# speedup vs baseline: 3.2580x; 1.0107x over previous
import functools

import jax
import jax.numpy as jnp
from jax import lax
from jax.experimental import pallas as pl
from jax.experimental.pallas import tpu as pltpu
from jax.experimental.pallas import tpu_sc as plsc

MAX_LEN = 200
N_FILTERS = 128
BATCH = 4096
SEQ = 200

_B = BATCH * SEQ
_NC = 2
_NS = 16
_NW = _NC * _NS
_PER_W = _B // _NW
_R = 400
_CHUNKS = _PER_W // _R


def _build_gather():
    mesh = plsc.VectorSubcoreMesh(core_axis_name="c", subcore_axis_name="s")

    @functools.partial(
        pl.kernel,
        mesh=mesh,
        out_type=jax.ShapeDtypeStruct((_B, N_FILTERS), jnp.float32),
        scratch_types=[
            pltpu.VMEM((_R,), jnp.int32),
            pltpu.VMEM((_R,), jnp.int32),
            pltpu.VMEM((_R, N_FILTERS), jnp.float32),
            pltpu.VMEM((_R, N_FILTERS), jnp.float32),
            pltpu.SemaphoreType.DMA,
            pltpu.SemaphoreType.DMA,
            pltpu.SemaphoreType.DMA,
            pltpu.SemaphoreType.DMA,
        ],
    )
    def gather_kernel(table_hbm, idx_hbm, out_hbm,
                      idx0, idx1, rows0, rows1, sg0, sg1, so0, so1):
        wid = lax.axis_index("s") * _NC + lax.axis_index("c")
        base0 = wid * _PER_W
        idx_b = (idx0, idx1)
        rows_b = (rows0, rows1)
        sg_b = (sg0, sg1)
        so_b = (so0, so1)

        def pair(j, carry):
            for b in range(2):
                i = 2 * j + b
                base = base0 + i * _R

                @pl.when(i >= 2)
                def _():
                    pltpu.make_async_copy(
                        rows_b[b], out_hbm.at[pl.ds(base - 2 * _R, _R)],
                        so_b[b]).wait()

                pltpu.sync_copy(idx_hbm.at[pl.ds(base, _R)], idx_b[b])
                pltpu.async_copy(table_hbm.at[idx_b[b]], rows_b[b],
                                 sg_b[b]).wait()
                pltpu.async_copy(rows_b[b], out_hbm.at[pl.ds(base, _R)],
                                 so_b[b])
            return carry

        lax.fori_loop(0, _CHUNKS // 2, pair, 0)

        for b in range(2):
            i = _CHUNKS - 2 + b
            pltpu.make_async_copy(
                rows_b[b], out_hbm.at[pl.ds(base0 + i * _R, _R)],
                so_b[b]).wait()

    return gather_kernel


_gather = _build_gather()


@jax.jit
def kernel(categories, ce):
    idx = categories.reshape(_B)
    out = _gather(ce, idx)
    return out.reshape(BATCH, SEQ, N_FILTERS)


# speedup vs baseline: 13.7829x; 4.2304x over previous
import functools

import jax
import jax.numpy as jnp
from jax import lax
from jax.experimental import pallas as pl
from jax.experimental.pallas import tpu as pltpu
from jax.experimental.pallas import tpu_sc as plsc

MAX_LEN = 200
N_FILTERS = 128
BATCH = 4096
SEQ = 200

_B = BATCH * SEQ
_NC = 2
_NS = 16
_NW = _NC * _NS
_PER_W = _B // _NW
_R = 400
_CHUNKS = _PER_W // _R


def _build_gather():
    mesh = plsc.VectorSubcoreMesh(core_axis_name="c", subcore_axis_name="s")

    @functools.partial(
        pl.kernel,
        mesh=mesh,
        out_type=jax.ShapeDtypeStruct((_B, N_FILTERS), jnp.float32),
        scratch_types=[
            pltpu.VMEM_SHARED((MAX_LEN, N_FILTERS), jnp.float32),
            pltpu.VMEM((_R,), jnp.int32),
            pltpu.VMEM((_R,), jnp.int32),
            pltpu.VMEM((_R, N_FILTERS), jnp.float32),
            pltpu.VMEM((_R, N_FILTERS), jnp.float32),
            pltpu.SemaphoreType.DMA,
            pltpu.SemaphoreType.DMA,
            pltpu.SemaphoreType.DMA,
            pltpu.SemaphoreType.DMA,
        ],
    )
    def gather_kernel(table_hbm, idx_hbm, out_hbm,
                      table_v, idx0, idx1, rows0, rows1, sg0, sg1, so0, so1):
        wid = lax.axis_index("s") * _NC + lax.axis_index("c")
        base0 = wid * _PER_W
        idx_b = (idx0, idx1)
        rows_b = (rows0, rows1)
        sg_b = (sg0, sg1)
        so_b = (so0, so1)

        @pl.when(lax.axis_index("s") == 0)
        def _():
            pltpu.sync_copy(table_hbm, table_v)

        plsc.subcore_barrier()

        def pair(j, carry):
            for b in range(2):
                i = 2 * j + b
                base = base0 + i * _R

                @pl.when(i >= 2)
                def _():
                    pltpu.make_async_copy(
                        rows_b[b], out_hbm.at[pl.ds(base - 2 * _R, _R)],
                        so_b[b]).wait()

                pltpu.sync_copy(idx_hbm.at[pl.ds(base, _R)], idx_b[b])
                pltpu.async_copy(table_v.at[idx_b[b]], rows_b[b],
                                 sg_b[b]).wait()
                pltpu.async_copy(rows_b[b], out_hbm.at[pl.ds(base, _R)],
                                 so_b[b])
            return carry

        lax.fori_loop(0, _CHUNKS // 2, pair, 0)

        for b in range(2):
            i = _CHUNKS - 2 + b
            pltpu.make_async_copy(
                rows_b[b], out_hbm.at[pl.ds(base0 + i * _R, _R)],
                so_b[b]).wait()

    return gather_kernel


_gather = _build_gather()


@jax.jit
def kernel(categories, ce):
    idx = categories.reshape(_B)
    out = _gather(ce, idx)
    return out.reshape(BATCH, SEQ, N_FILTERS)


# speedup vs baseline: 15.5152x; 1.1257x over previous
import functools

import jax
import jax.numpy as jnp
from jax import lax
from jax.experimental import pallas as pl
from jax.experimental.pallas import tpu as pltpu
from jax.experimental.pallas import tpu_sc as plsc

MAX_LEN = 200
N_FILTERS = 128
BATCH = 4096
SEQ = 200

_B = BATCH * SEQ
_NC = 2
_NS = 16
_NW = _NC * _NS
_PER_W = _B // _NW
_R = 400
_CHUNKS = _PER_W // _R


def _build_gather():
    mesh = plsc.VectorSubcoreMesh(core_axis_name="c", subcore_axis_name="s")

    @functools.partial(
        pl.kernel,
        mesh=mesh,
        out_type=jax.ShapeDtypeStruct((_B, N_FILTERS), jnp.float32),
        scratch_types=[
            pltpu.VMEM_SHARED((MAX_LEN, N_FILTERS), jnp.float32),
            pltpu.VMEM((_R,), jnp.int32),
            pltpu.VMEM((_R,), jnp.int32),
            pltpu.VMEM((_R, N_FILTERS), jnp.float32),
            pltpu.VMEM((_R, N_FILTERS), jnp.float32),
            pltpu.SemaphoreType.DMA,
            pltpu.SemaphoreType.DMA,
            pltpu.SemaphoreType.DMA,
            pltpu.SemaphoreType.DMA,
            pltpu.SemaphoreType.DMA,
            pltpu.SemaphoreType.DMA,
        ],
    )
    def gather_kernel(table_hbm, idx_hbm, out_hbm,
                      table_sh, idx0, idx1, rows0, rows1,
                      si0, si1, sg0, sg1, so0, so1):
        wid = lax.axis_index("s") * _NC + lax.axis_index("c")
        base0 = wid * _PER_W
        idx_b = (idx0, idx1)
        rows_b = (rows0, rows1)
        si_b = (si0, si1)
        sg_b = (sg0, sg1)
        so_b = (so0, so1)

        @pl.when(lax.axis_index("s") == 0)
        def _():
            pltpu.sync_copy(table_hbm, table_sh)

        plsc.subcore_barrier()

        pltpu.async_copy(idx_hbm.at[pl.ds(base0, _R)], idx0, si0)

        def pair(j, carry):
            for b in range(2):
                i = 2 * j + b
                nb = 1 - b
                base = base0 + i * _R

                @pl.when(i + 1 < _CHUNKS)
                def _():
                    pltpu.async_copy(
                        idx_hbm.at[pl.ds(base + _R, _R)], idx_b[nb],
                        si_b[nb])

                pltpu.make_async_copy(
                    idx_hbm.at[pl.ds(base, _R)], idx_b[b], si_b[b]).wait()

                @pl.when(i >= 2)
                def _():
                    pltpu.make_async_copy(
                        rows_b[b], out_hbm.at[pl.ds(base - 2 * _R, _R)],
                        so_b[b]).wait()

                pltpu.async_copy(table_sh.at[idx_b[b]], rows_b[b],
                                 sg_b[b]).wait()
                pltpu.async_copy(rows_b[b], out_hbm.at[pl.ds(base, _R)],
                                 so_b[b])
            return carry

        lax.fori_loop(0, _CHUNKS // 2, pair, 0)

        for b in range(2):
            i = _CHUNKS - 2 + b
            pltpu.make_async_copy(
                rows_b[b], out_hbm.at[pl.ds(base0 + i * _R, _R)],
                so_b[b]).wait()

    return gather_kernel


_gather = _build_gather()


@jax.jit
def kernel(categories, ce):
    idx = categories.reshape(_B)
    out = _gather(ce, idx)
    return out.reshape(BATCH, SEQ, N_FILTERS)


# speedup vs baseline: 15.5390x; 1.0015x over previous
import functools

import jax
import jax.numpy as jnp
from jax import lax
from jax.experimental import pallas as pl
from jax.experimental.pallas import tpu as pltpu
from jax.experimental.pallas import tpu_sc as plsc

MAX_LEN = 200
N_FILTERS = 128
BATCH = 4096
SEQ = 200

_B = BATCH * SEQ
_NC = 2
_NS = 16
_NW = _NC * _NS
_PER_W = _B // _NW
_R = 400
_CHUNKS = _PER_W // _R


def _build_gather():
    mesh = plsc.VectorSubcoreMesh(core_axis_name="c", subcore_axis_name="s")

    @functools.partial(
        pl.kernel,
        mesh=mesh,
        out_type=jax.ShapeDtypeStruct((_B, N_FILTERS), jnp.float32),
        scratch_types=[
            pltpu.VMEM_SHARED((MAX_LEN, N_FILTERS), jnp.float32),
            pltpu.VMEM((_R,), jnp.int32),
            pltpu.VMEM((_R,), jnp.int32),
            pltpu.VMEM((_R, N_FILTERS), jnp.float32),
            pltpu.VMEM((_R, N_FILTERS), jnp.float32),
            pltpu.SemaphoreType.DMA,
            pltpu.SemaphoreType.DMA,
            pltpu.SemaphoreType.DMA,
            pltpu.SemaphoreType.DMA,
            pltpu.SemaphoreType.DMA,
            pltpu.SemaphoreType.DMA,
        ],
    )
    def gather_kernel(table_hbm, idx_hbm, out_hbm,
                      table_sh, idx0, idx1, rows0, rows1,
                      si0, si1, sg0, sg1, so0, so1):
        wid = lax.axis_index("s") * _NC + lax.axis_index("c")
        base0 = wid * _PER_W
        idx_b = (idx0, idx1)
        rows_b = (rows0, rows1)
        si_b = (si0, si1)
        sg_b = (sg0, sg1)
        so_b = (so0, so1)

        def idx_slice(i):
            return idx_hbm.at[pl.ds(base0 + i * _R, _R)]

        def out_slice(i):
            return out_hbm.at[pl.ds(base0 + i * _R, _R)]

        @pl.when(lax.axis_index("s") == 0)
        def _():
            pltpu.sync_copy(table_hbm, table_sh)

        plsc.subcore_barrier()

        pltpu.async_copy(idx_slice(0), idx0, si0)
        pltpu.async_copy(idx_slice(1), idx1, si1)
        pltpu.make_async_copy(idx_slice(0), idx0, si0).wait()
        pltpu.async_copy(table_sh.at[idx0], rows0, sg0)

        def pair(j, carry):
            for b in range(2):
                i = 2 * j + b
                nb = 1 - b

                @pl.when(i >= 1)
                def _():
                    pltpu.make_async_copy(rows_b[nb], out_slice(i - 1),
                                          so_b[nb]).wait()

                @pl.when(i + 1 < _CHUNKS)
                def _():
                    pltpu.make_async_copy(idx_slice(i + 1), idx_b[nb],
                                          si_b[nb]).wait()
                    pltpu.async_copy(table_sh.at[idx_b[nb]], rows_b[nb],
                                     sg_b[nb])

                pltpu.make_async_copy(table_sh.at[idx_b[b]], rows_b[b],
                                      sg_b[b]).wait()

                @pl.when(i + 2 < _CHUNKS)
                def _():
                    pltpu.async_copy(idx_slice(i + 2), idx_b[b], si_b[b])

                pltpu.async_copy(rows_b[b], out_slice(i), so_b[b])
            return carry

        lax.fori_loop(0, _CHUNKS // 2, pair, 0)

        pltpu.make_async_copy(rows_b[1], out_slice(_CHUNKS - 1),
                              so_b[1]).wait()

    return gather_kernel


_gather = _build_gather()


@jax.jit
def kernel(categories, ce):
    idx = categories.reshape(_B)
    out = _gather(ce, idx)
    return out.reshape(BATCH, SEQ, N_FILTERS)
